# baseline (speedup 1.0000x reference)
import jax
import jax.numpy as jnp
from jax import lax
from jax.experimental import pallas as pl
from jax.experimental.pallas import tpu as pltpu
from jax.experimental.pallas import tpu_sc as plsc

B, S, H = 1, 2048, 1024
FF = 4096
E = 8
NEXP = 10
EPAD = 16
TM = 256
A = 2 * S
MT = 256
PADTOT = A + E * MT
NTILES = PADTOT // MT
FFB = 1024
NF = FF // FFB
NC, NS = 2, 16
NW = NC * NS
CHUNK = A // NS
GROWS = PADTOT // NW
TRASH = PADTOT
SORTN = 6400


def _router_body(x_ref, g_ref, logits_ref, eslots_ref, wslots_ref,
                 pos8_ref, eotvld_ref, sb0_ref, sb1_ref, srun_ref, soffs_ref):
    p = pl.program_id(0)
    t = pl.program_id(1)
    x = x_ref[...].astype(jnp.bfloat16)
    g = g_ref[...].astype(jnp.bfloat16)
    l = jax.lax.dot_general(x, g, (((1,), (1,)), ((), ())),
                            preferred_element_type=jnp.float32)
    logits_ref[...] = l

    cols = jax.lax.broadcasted_iota(jnp.int32, (TM, EPAD), 1)
    lm = jnp.where(cols < NEXP, l, -1e30)
    m = jnp.max(lm, axis=1, keepdims=True)
    pr = jnp.exp(lm - m)
    prob = pr / jnp.sum(pr, axis=1, keepdims=True)

    p1 = jnp.max(prob, axis=1, keepdims=True)
    a1 = jnp.min(jnp.where(prob == p1, cols, 999), axis=1, keepdims=True)
    prob2 = jnp.where(cols == a1, -1.0, prob)
    p2 = jnp.max(prob2, axis=1, keepdims=True)
    a2 = jnp.min(jnp.where(prob2 == p2, cols, 999), axis=1, keepdims=True)

    m1 = (a1 < E).astype(jnp.float32)
    m2 = (a2 < E).astype(jnp.float32)
    sw1 = p1 * m1
    sw2 = p2 * m2
    ssum = sw1 + sw2
    ssum = jnp.where(ssum == 0.0, 1.0, ssum)
    w1 = sw1 / ssum
    w2 = sw2 / ssum
    c8 = jax.lax.broadcasted_iota(jnp.int32, (TM, 8), 1)
    eslots_ref[...] = jnp.where(c8 == 0, a1, jnp.where(c8 == 1, a2, 0))
    wslots_ref[...] = jnp.where(c8 == 0, w1, jnp.where(c8 == 1, w2, 0.0))

    i1 = (cols == a1).astype(jnp.float32)
    i2 = (cols == a2).astype(jnp.float32)

    @pl.when(p == 0)
    def _pass0():
        cnt1 = jnp.sum(i1, axis=0, keepdims=True)
        cnt2 = jnp.sum(i2, axis=0, keepdims=True)

        @pl.when(t == 0)
        def _init():
            srun_ref[...] = jnp.zeros((2, EPAD), jnp.float32)

        sb0_ref[t, :] = srun_ref[0, :]
        sb1_ref[t, :] = srun_ref[1, :]
        srun_ref[0, :] += cnt1[0, :]
        srun_ref[1, :] += cnt2[0, :]

        @pl.when(t == pl.num_programs(1) - 1)
        def _totals():
            lane = jax.lax.broadcasted_iota(jnp.int32, (1, EPAD), 1)
            tot = (srun_ref[0, :] + srun_ref[1, :]).reshape(1, EPAD)
            tot = tot * (lane < E).astype(jnp.float32)
            padded = jnp.floor((tot + (MT - 1)) / MT) * MT
            li = jax.lax.broadcasted_iota(jnp.int32, (EPAD, EPAD), 0)
            lj = jax.lax.broadcasted_iota(jnp.int32, (EPAD, EPAD), 1)
            lstrict = (li < lj).astype(jnp.float32)
            lincl = (li <= lj).astype(jnp.float32)
            offs = jax.lax.dot_general(padded, lstrict,
                                       (((1,), (0,)), ((), ())),
                                       preferred_element_type=jnp.float32)
            soffs_ref[...] = offs
            cum_tiles = jax.lax.dot_general(padded / MT, lincl,
                                            (((1,), (0,)), ((), ())),
                                            preferred_element_type=jnp.float32)
            n_tiles = jnp.sum(padded) / MT
            tc = jax.lax.broadcasted_iota(
                jnp.int32, (8, 128), 1).astype(jnp.float32)
            acc = jnp.zeros((8, 128), jnp.float32)
            for e in range(E):
                ct_e = jnp.sum(cum_tiles * (lane == e).astype(jnp.float32))
                acc = acc + (tc >= ct_e).astype(jnp.float32)
            eotrow = jnp.minimum(acc, float(E - 1))
            vldrow = (tc < n_tiles).astype(jnp.float32)
            rr = jax.lax.broadcasted_iota(jnp.int32, (8, 128), 0)
            eotvld_ref[...] = jnp.where(
                rr == 0, eotrow,
                jnp.where(rr == 1, vldrow, 0.0)).astype(jnp.int32)

    @pl.when(p == 1)
    def _pass1():
        ri = jax.lax.broadcasted_iota(jnp.int32, (TM, TM), 0)
        rj = jax.lax.broadcasted_iota(jnp.int32, (TM, TM), 1)
        trils = (rj < ri).astype(jnp.float32)
        r1 = jax.lax.dot_general(trils, i1, (((1,), (0,)), ((), ())),
                                 preferred_element_type=jnp.float32)
        r2 = jax.lax.dot_general(trils, i2, (((1,), (0,)), ((), ())),
                                 preferred_element_type=jnp.float32)
        offs = soffs_ref[...]
        tot0 = srun_ref[0, :].reshape(1, EPAD)
        b1 = offs + sb0_ref[t, :].reshape(1, EPAD)
        b2 = offs + tot0 + sb1_ref[t, :].reshape(1, EPAD)
        pos1 = jnp.sum((r1 + b1) * i1, axis=1, keepdims=True)
        pos2 = jnp.sum((r2 + b2) * i2, axis=1, keepdims=True)
        m1f = (a1 < E).astype(jnp.float32)
        m2f = (a2 < E).astype(jnp.float32)
        p1v = (pos1 * m1f).astype(jnp.int32)
        p2v = (pos2 * m2f).astype(jnp.int32)
        s1v = jnp.where(a1 < E, pos1.astype(jnp.int32), TRASH)
        s2v = jnp.where(a2 < E, pos2.astype(jnp.int32), TRASH)
        pos8_ref[...] = jnp.where(
            c8 == 0, p1v,
            jnp.where(c8 == 1, p2v,
                      jnp.where(c8 == 2, s1v,
                                jnp.where(c8 == 3, s2v, 0))))


def _run_router(x, gpad):
    nt = S // TM
    return pl.pallas_call(
        _router_body,
        grid=(2, nt),
        in_specs=[
            pl.BlockSpec((TM, H), lambda p, t: (t, 0)),
            pl.BlockSpec((EPAD, H), lambda p, t: (0, 0)),
        ],
        out_specs=[
            pl.BlockSpec((TM, EPAD), lambda p, t: (t, 0)),
            pl.BlockSpec((TM, 8), lambda p, t: (t, 0)),
            pl.BlockSpec((TM, 8), lambda p, t: (t, 0)),
            pl.BlockSpec((TM, 8), lambda p, t: (t, 0)),
            pl.BlockSpec((8, 128), lambda p, t: (0, 0)),
        ],
        out_shape=[
            jax.ShapeDtypeStruct((S, EPAD), jnp.float32),
            jax.ShapeDtypeStruct((S, 8), jnp.int32),
            jax.ShapeDtypeStruct((S, 8), jnp.float32),
            jax.ShapeDtypeStruct((S, 8), jnp.int32),
            jax.ShapeDtypeStruct((8, 128), jnp.int32),
        ],
        scratch_shapes=[
            pltpu.VMEM((8, EPAD), jnp.float32),
            pltpu.VMEM((8, EPAD), jnp.float32),
            pltpu.VMEM((2, EPAD), jnp.float32),
            pltpu.VMEM((1, EPAD), jnp.float32),
        ],
    )(x, gpad)


def _dispatch_body(sc_hbm, tids_hbm, x_hbm, xs_hbm,
                   sc_a, sc_b, tid_a, tid_b, zero_v, idxg_v,
                   rows_a, rows_b, sema, semb, sort_sp):
    c = lax.axis_index("c")
    w = lax.axis_index("s")
    g = w * NC + c
    CH = GROWS // 4

    zl = SORTN // NS
    for q in range(zl // 16):
        zero_v[pl.ds(q * 16, 16)] = jnp.zeros((16,), jnp.int32)
    pltpu.sync_copy(zero_v, sort_sp.at[pl.ds(w * zl, zl)])
    plsc.subcore_barrier()

    pltpu.sync_copy(sc_hbm.at[pl.ds(w * CHUNK, 128)], sc_a)
    pltpu.sync_copy(sc_hbm.at[pl.ds(w * CHUNK + 128, 128)], sc_b)
    pltpu.sync_copy(tids_hbm.at[pl.ds(w * CHUNK, 128)], tid_a)
    pltpu.sync_copy(tids_hbm.at[pl.ds(w * CHUNK + 128, 128)], tid_b)
    pltpu.sync_copy(tid_a, sort_sp.at[sc_a])
    pltpu.sync_copy(tid_b, sort_sp.at[sc_b])
    plsc.subcore_barrier()

    base = g * GROWS
    pltpu.sync_copy(sort_sp.at[pl.ds(base, GROWS)], idxg_v)
    d0 = pltpu.async_copy(x_hbm.at[idxg_v.at[pl.ds(0, CH)]], rows_a, sema)
    d1 = pltpu.async_copy(x_hbm.at[idxg_v.at[pl.ds(CH, CH)]], rows_b, semb)
    d0.wait()
    pltpu.sync_copy(rows_a, xs_hbm.at[pl.ds(base, CH)])
    d2 = pltpu.async_copy(x_hbm.at[idxg_v.at[pl.ds(2 * CH, CH)]], rows_a, sema)
    d1.wait()
    pltpu.sync_copy(rows_b, xs_hbm.at[pl.ds(base + CH, CH)])
    d3 = pltpu.async_copy(x_hbm.at[idxg_v.at[pl.ds(3 * CH, CH)]], rows_b, semb)
    d2.wait()
    pltpu.sync_copy(rows_a, xs_hbm.at[pl.ds(base + 2 * CH, CH)])
    d3.wait()
    pltpu.sync_copy(rows_b, xs_hbm.at[pl.ds(base + 3 * CH, CH)])


def _run_dispatch(sc_flat, tids, x):
    mesh = plsc.VectorSubcoreMesh(core_axis_name="c", subcore_axis_name="s",
                                  num_cores=NC, num_subcores=NS)
    f = pl.kernel(
        _dispatch_body,
        out_type=jax.ShapeDtypeStruct((PADTOT, H), jnp.float32),
        mesh=mesh,
        scratch_types=[
            pltpu.VMEM((128,), jnp.int32),
            pltpu.VMEM((128,), jnp.int32),
            pltpu.VMEM((128,), jnp.int32),
            pltpu.VMEM((128,), jnp.int32),
            pltpu.VMEM((SORTN // NS,), jnp.int32),
            pltpu.VMEM((GROWS,), jnp.int32),
            pltpu.VMEM((GROWS // 4, H), jnp.float32),
            pltpu.VMEM((GROWS // 4, H), jnp.float32),
            pltpu.SemaphoreType.DMA,
            pltpu.SemaphoreType.DMA,
            pltpu.VMEM_SHARED((SORTN,), jnp.int32),
        ],
    )
    return f(sc_flat, tids, x)


def _ffn_body(eot_ref, vld_ref, xs_ref, w1_ref, w3_ref, w2_ref, out_ref,
              acc_ref):
    f = pl.program_id(0)
    i = pl.program_id(1)

    @pl.when(jnp.logical_and(vld_ref[i] == 0, f == NF - 1))
    def _zero():
        out_ref[...] = jnp.zeros((MT, H), jnp.float32)

    @pl.when(vld_ref[i] == 1)
    def _go():
        x = xs_ref[...].astype(jnp.bfloat16)
        a = jax.lax.dot_general(x, w1_ref[0], (((1,), (1,)), ((), ())),
                                preferred_element_type=jnp.float32)
        b = jax.lax.dot_general(x, w3_ref[0], (((1,), (1,)), ((), ())),
                                preferred_element_type=jnp.float32)
        h1 = (a * jax.nn.sigmoid(a) * b).astype(jnp.bfloat16)
        o = jax.lax.dot_general(h1, w2_ref[0], (((1,), (1,)), ((), ())),
                                preferred_element_type=jnp.float32)
        sl = pl.ds(i * MT, MT)

        @pl.when(f == 0)
        def _init():
            acc_ref[sl, :] = o

        @pl.when(f > 0)
        def _acc():
            acc_ref[sl, :] += o

        @pl.when(f == NF - 1)
        def _emit():
            out_ref[...] = acc_ref[sl, :]


def _run_ffn(eot, vld, xs, W1b, W3b, W2b):
    grid_spec = pltpu.PrefetchScalarGridSpec(
        num_scalar_prefetch=2,
        grid=(NF, NTILES),
        in_specs=[
            pl.BlockSpec((MT, H), lambda f, i, eot, vld: (i, 0)),
            pl.BlockSpec((1, FFB, H), lambda f, i, eot, vld: (eot[i], f, 0)),
            pl.BlockSpec((1, FFB, H), lambda f, i, eot, vld: (eot[i], f, 0)),
            pl.BlockSpec((1, H, FFB), lambda f, i, eot, vld: (eot[i], 0, f)),
        ],
        out_specs=pl.BlockSpec((MT, H), lambda f, i, eot, vld: (i, 0)),
        scratch_shapes=[pltpu.VMEM((PADTOT, H), jnp.float32)],
    )
    return pl.pallas_call(
        _ffn_body,
        grid_spec=grid_spec,
        out_shape=jax.ShapeDtypeStruct((PADTOT, H), jnp.float32),
    )(eot, vld, xs, W1b, W3b, W2b)


def _gather2_body(os_hbm, p0_hbm, p1_hbm, y0_hbm, y1_hbm,
                  idx0_v, idx1_v, ra, rb, sema, semb):
    c = lax.axis_index("c")
    w = lax.axis_index("s")
    g = w * NC + c
    tpw = S // NW
    gb = g * tpw
    hh = tpw // 2
    pltpu.sync_copy(p0_hbm.at[pl.ds(gb, tpw)], idx0_v)
    pltpu.sync_copy(p1_hbm.at[pl.ds(gb, tpw)], idx1_v)
    d0 = pltpu.async_copy(os_hbm.at[idx0_v.at[pl.ds(0, hh)]], ra, sema)
    d1 = pltpu.async_copy(os_hbm.at[idx0_v.at[pl.ds(hh, hh)]], rb, semb)
    d0.wait()
    pltpu.sync_copy(ra, y0_hbm.at[pl.ds(gb, hh)])
    d2 = pltpu.async_copy(os_hbm.at[idx1_v.at[pl.ds(0, hh)]], ra, sema)
    d1.wait()
    pltpu.sync_copy(rb, y0_hbm.at[pl.ds(gb + hh, hh)])
    d3 = pltpu.async_copy(os_hbm.at[idx1_v.at[pl.ds(hh, hh)]], rb, semb)
    d2.wait()
    pltpu.sync_copy(ra, y1_hbm.at[pl.ds(gb, hh)])
    d3.wait()
    pltpu.sync_copy(rb, y1_hbm.at[pl.ds(gb + hh, hh)])


def _run_gather2(out_sorted, pos0, pos1):
    mesh = plsc.VectorSubcoreMesh(core_axis_name="c", subcore_axis_name="s",
                                  num_cores=NC, num_subcores=NS)
    f = pl.kernel(
        _gather2_body,
        out_type=[
            jax.ShapeDtypeStruct((S, H), jnp.float32),
            jax.ShapeDtypeStruct((S, H), jnp.float32),
        ],
        mesh=mesh,
        scratch_types=[
            pltpu.VMEM((S // NW,), jnp.int32),
            pltpu.VMEM((S // NW,), jnp.int32),
            pltpu.VMEM((S // NW // 2, H), jnp.float32),
            pltpu.VMEM((S // NW // 2, H), jnp.float32),
            pltpu.SemaphoreType.DMA,
            pltpu.SemaphoreType.DMA,
        ],
    )
    return f(out_sorted, pos0, pos1)


def _comb_body(ws_ref, y0_ref, y1_ref, out_ref):
    ws = ws_ref[...]
    out_ref[...] = (y0_ref[...] * ws[:, 0:1] + y1_ref[...] * ws[:, 1:2])


def _run_comb(wslots, y0, y1):
    nt = S // TM
    return pl.pallas_call(
        _comb_body,
        grid=(nt,),
        in_specs=[
            pl.BlockSpec((TM, 8), lambda t: (t, 0)),
            pl.BlockSpec((TM, H), lambda t: (t, 0)),
            pl.BlockSpec((TM, H), lambda t: (t, 0)),
        ],
        out_specs=pl.BlockSpec((TM, H), lambda t: (t, 0)),
        out_shape=jax.ShapeDtypeStruct((S, H), jnp.float32),
    )(wslots, y0, y1)


def kernel(hidden_states, gate_w, gate2_w, W1, W2, W3):
    x = hidden_states.reshape(-1, H)
    gpad = jnp.zeros((EPAD, H), jnp.float32).at[:NEXP].set(
        jnp.concatenate([gate_w, gate2_w], axis=0))

    logits16, eslots, wslots, pos8, eotvld = _run_router(x, gpad)

    sc_flat = jnp.concatenate([pos8[:, 2], pos8[:, 3]])
    tids = jnp.arange(A, dtype=jnp.int32) % S
    xs = _run_dispatch(sc_flat, tids, x)

    W1b = W1.astype(jnp.bfloat16)
    W2b = W2.astype(jnp.bfloat16)
    W3b = W3.astype(jnp.bfloat16)
    eot = eotvld[0, :NTILES]
    vld = eotvld[1, :NTILES]
    out_sorted = _run_ffn(eot, vld, xs, W1b, W3b, W2b)

    y0, y1 = _run_gather2(out_sorted, pos8[:, 0], pos8[:, 1])
    final = _run_comb(wslots, y0, y1)
    return final.reshape(B, S, H), logits16[:, :NEXP]


# speedup vs baseline: 1.0781x; 1.0781x over previous
import jax
import jax.numpy as jnp
from jax import lax
from jax.experimental import pallas as pl
from jax.experimental.pallas import tpu as pltpu
from jax.experimental.pallas import tpu_sc as plsc

B, S, H = 1, 2048, 1024
FF = 4096
E = 8
NEXP = 10
EPAD = 16
TM = 256
A = 2 * S
MT = 256
PADTOT = A + E * MT
NTILES = PADTOT // MT
FFB = 2048
NF = FF // FFB
NC, NS = 2, 16
NW = NC * NS
CHUNK = A // NS
GROWS = PADTOT // NW
TRASH = PADTOT
SORTN = 6400


def _router_body(x_ref, g_ref, logits_ref, eslots_ref, wslots_ref,
                 pos8_ref, eotvld_ref, sb0_ref, sb1_ref, srun_ref, soffs_ref):
    p = pl.program_id(0)
    t = pl.program_id(1)
    x = x_ref[...].astype(jnp.bfloat16)
    g = g_ref[...].astype(jnp.bfloat16)
    l = jax.lax.dot_general(x, g, (((1,), (1,)), ((), ())),
                            preferred_element_type=jnp.float32)
    logits_ref[...] = l

    cols = jax.lax.broadcasted_iota(jnp.int32, (TM, EPAD), 1)
    lm = jnp.where(cols < NEXP, l, -1e30)
    m = jnp.max(lm, axis=1, keepdims=True)
    pr = jnp.exp(lm - m)
    prob = pr / jnp.sum(pr, axis=1, keepdims=True)

    p1 = jnp.max(prob, axis=1, keepdims=True)
    a1 = jnp.min(jnp.where(prob == p1, cols, 999), axis=1, keepdims=True)
    prob2 = jnp.where(cols == a1, -1.0, prob)
    p2 = jnp.max(prob2, axis=1, keepdims=True)
    a2 = jnp.min(jnp.where(prob2 == p2, cols, 999), axis=1, keepdims=True)

    m1 = (a1 < E).astype(jnp.float32)
    m2 = (a2 < E).astype(jnp.float32)
    sw1 = p1 * m1
    sw2 = p2 * m2
    ssum = sw1 + sw2
    ssum = jnp.where(ssum == 0.0, 1.0, ssum)
    w1 = sw1 / ssum
    w2 = sw2 / ssum
    c8 = jax.lax.broadcasted_iota(jnp.int32, (TM, 8), 1)
    eslots_ref[...] = jnp.where(c8 == 0, a1, jnp.where(c8 == 1, a2, 0))
    wslots_ref[...] = jnp.where(c8 == 0, w1, jnp.where(c8 == 1, w2, 0.0))

    i1 = (cols == a1).astype(jnp.float32)
    i2 = (cols == a2).astype(jnp.float32)

    @pl.when(p == 0)
    def _pass0():
        cnt1 = jnp.sum(i1, axis=0, keepdims=True)
        cnt2 = jnp.sum(i2, axis=0, keepdims=True)

        @pl.when(t == 0)
        def _init():
            srun_ref[...] = jnp.zeros((2, EPAD), jnp.float32)

        sb0_ref[t, :] = srun_ref[0, :]
        sb1_ref[t, :] = srun_ref[1, :]
        srun_ref[0, :] += cnt1[0, :]
        srun_ref[1, :] += cnt2[0, :]

        @pl.when(t == pl.num_programs(1) - 1)
        def _totals():
            lane = jax.lax.broadcasted_iota(jnp.int32, (1, EPAD), 1)
            tot = (srun_ref[0, :] + srun_ref[1, :]).reshape(1, EPAD)
            tot = tot * (lane < E).astype(jnp.float32)
            padded = jnp.floor((tot + (MT - 1)) / MT) * MT
            li = jax.lax.broadcasted_iota(jnp.int32, (EPAD, EPAD), 0)
            lj = jax.lax.broadcasted_iota(jnp.int32, (EPAD, EPAD), 1)
            lstrict = (li < lj).astype(jnp.float32)
            lincl = (li <= lj).astype(jnp.float32)
            offs = jax.lax.dot_general(padded, lstrict,
                                       (((1,), (0,)), ((), ())),
                                       preferred_element_type=jnp.float32)
            soffs_ref[...] = offs
            cum_tiles = jax.lax.dot_general(padded / MT, lincl,
                                            (((1,), (0,)), ((), ())),
                                            preferred_element_type=jnp.float32)
            n_tiles = jnp.sum(padded) / MT
            tc = jax.lax.broadcasted_iota(
                jnp.int32, (8, 128), 1).astype(jnp.float32)
            acc = jnp.zeros((8, 128), jnp.float32)
            for e in range(E):
                ct_e = jnp.sum(cum_tiles * (lane == e).astype(jnp.float32))
                acc = acc + (tc >= ct_e).astype(jnp.float32)
            eotrow = jnp.minimum(acc, float(E - 1))
            vldrow = (tc < n_tiles).astype(jnp.float32)
            rr = jax.lax.broadcasted_iota(jnp.int32, (8, 128), 0)
            eotvld_ref[...] = jnp.where(
                rr == 0, eotrow,
                jnp.where(rr == 1, vldrow, 0.0)).astype(jnp.int32)

    @pl.when(p == 1)
    def _pass1():
        ri = jax.lax.broadcasted_iota(jnp.int32, (TM, TM), 0)
        rj = jax.lax.broadcasted_iota(jnp.int32, (TM, TM), 1)
        trils = (rj < ri).astype(jnp.float32)
        r1 = jax.lax.dot_general(trils, i1, (((1,), (0,)), ((), ())),
                                 preferred_element_type=jnp.float32)
        r2 = jax.lax.dot_general(trils, i2, (((1,), (0,)), ((), ())),
                                 preferred_element_type=jnp.float32)
        offs = soffs_ref[...]
        tot0 = srun_ref[0, :].reshape(1, EPAD)
        b1 = offs + sb0_ref[t, :].reshape(1, EPAD)
        b2 = offs + tot0 + sb1_ref[t, :].reshape(1, EPAD)
        pos1 = jnp.sum((r1 + b1) * i1, axis=1, keepdims=True)
        pos2 = jnp.sum((r2 + b2) * i2, axis=1, keepdims=True)
        m1f = (a1 < E).astype(jnp.float32)
        m2f = (a2 < E).astype(jnp.float32)
        p1v = (pos1 * m1f).astype(jnp.int32)
        p2v = (pos2 * m2f).astype(jnp.int32)
        s1v = jnp.where(a1 < E, pos1.astype(jnp.int32), TRASH)
        s2v = jnp.where(a2 < E, pos2.astype(jnp.int32), TRASH)
        pos8_ref[...] = jnp.where(
            c8 == 0, p1v,
            jnp.where(c8 == 1, p2v,
                      jnp.where(c8 == 2, s1v,
                                jnp.where(c8 == 3, s2v, 0))))


def _run_router(x, gpad):
    nt = S // TM
    return pl.pallas_call(
        _router_body,
        grid=(2, nt),
        in_specs=[
            pl.BlockSpec((TM, H), lambda p, t: (t, 0)),
            pl.BlockSpec((EPAD, H), lambda p, t: (0, 0)),
        ],
        out_specs=[
            pl.BlockSpec((TM, EPAD), lambda p, t: (t, 0)),
            pl.BlockSpec((TM, 8), lambda p, t: (t, 0)),
            pl.BlockSpec((TM, 8), lambda p, t: (t, 0)),
            pl.BlockSpec((TM, 8), lambda p, t: (t, 0)),
            pl.BlockSpec((8, 128), lambda p, t: (0, 0)),
        ],
        out_shape=[
            jax.ShapeDtypeStruct((S, EPAD), jnp.float32),
            jax.ShapeDtypeStruct((S, 8), jnp.int32),
            jax.ShapeDtypeStruct((S, 8), jnp.float32),
            jax.ShapeDtypeStruct((S, 8), jnp.int32),
            jax.ShapeDtypeStruct((8, 128), jnp.int32),
        ],
        scratch_shapes=[
            pltpu.VMEM((8, EPAD), jnp.float32),
            pltpu.VMEM((8, EPAD), jnp.float32),
            pltpu.VMEM((2, EPAD), jnp.float32),
            pltpu.VMEM((1, EPAD), jnp.float32),
        ],
    )(x, gpad)


def _dispatch_body(sc_hbm, tids_hbm, x_hbm, xs_hbm,
                   sc_a, sc_b, tid_a, tid_b, zero_v, idxg_v,
                   rows_a, rows_b, sema, semb, sort_sp):
    c = lax.axis_index("c")
    w = lax.axis_index("s")
    g = w * NC + c
    CH = GROWS // 4

    zl = SORTN // NS
    for q in range(zl // 16):
        zero_v[pl.ds(q * 16, 16)] = jnp.zeros((16,), jnp.int32)
    pltpu.sync_copy(zero_v, sort_sp.at[pl.ds(w * zl, zl)])
    plsc.subcore_barrier()

    pltpu.sync_copy(sc_hbm.at[pl.ds(w * CHUNK, 128)], sc_a)
    pltpu.sync_copy(sc_hbm.at[pl.ds(w * CHUNK + 128, 128)], sc_b)
    pltpu.sync_copy(tids_hbm.at[pl.ds(w * CHUNK, 128)], tid_a)
    pltpu.sync_copy(tids_hbm.at[pl.ds(w * CHUNK + 128, 128)], tid_b)
    pltpu.sync_copy(tid_a, sort_sp.at[sc_a])
    pltpu.sync_copy(tid_b, sort_sp.at[sc_b])
    plsc.subcore_barrier()

    base = g * GROWS
    pltpu.sync_copy(sort_sp.at[pl.ds(base, GROWS)], idxg_v)
    d0 = pltpu.async_copy(x_hbm.at[idxg_v.at[pl.ds(0, CH)]], rows_a, sema)
    d1 = pltpu.async_copy(x_hbm.at[idxg_v.at[pl.ds(CH, CH)]], rows_b, semb)
    d0.wait()
    pltpu.sync_copy(rows_a, xs_hbm.at[pl.ds(base, CH)])
    d2 = pltpu.async_copy(x_hbm.at[idxg_v.at[pl.ds(2 * CH, CH)]], rows_a, sema)
    d1.wait()
    pltpu.sync_copy(rows_b, xs_hbm.at[pl.ds(base + CH, CH)])
    d3 = pltpu.async_copy(x_hbm.at[idxg_v.at[pl.ds(3 * CH, CH)]], rows_b, semb)
    d2.wait()
    pltpu.sync_copy(rows_a, xs_hbm.at[pl.ds(base + 2 * CH, CH)])
    d3.wait()
    pltpu.sync_copy(rows_b, xs_hbm.at[pl.ds(base + 3 * CH, CH)])


def _run_dispatch(sc_flat, tids, x):
    mesh = plsc.VectorSubcoreMesh(core_axis_name="c", subcore_axis_name="s",
                                  num_cores=NC, num_subcores=NS)
    f = pl.kernel(
        _dispatch_body,
        out_type=jax.ShapeDtypeStruct((PADTOT, H), jnp.float32),
        mesh=mesh,
        scratch_types=[
            pltpu.VMEM((128,), jnp.int32),
            pltpu.VMEM((128,), jnp.int32),
            pltpu.VMEM((128,), jnp.int32),
            pltpu.VMEM((128,), jnp.int32),
            pltpu.VMEM((SORTN // NS,), jnp.int32),
            pltpu.VMEM((GROWS,), jnp.int32),
            pltpu.VMEM((GROWS // 4, H), jnp.float32),
            pltpu.VMEM((GROWS // 4, H), jnp.float32),
            pltpu.SemaphoreType.DMA,
            pltpu.SemaphoreType.DMA,
            pltpu.VMEM_SHARED((SORTN,), jnp.int32),
        ],
    )
    return f(sc_flat, tids, x)


def _ffn_body(eot_ref, vld_ref, xs_ref, w1_ref, w3_ref, w2_ref, out_ref,
              acc_ref):
    f = pl.program_id(0)
    i = pl.program_id(1)

    @pl.when(jnp.logical_and(vld_ref[i] == 0, f == NF - 1))
    def _zero():
        out_ref[...] = jnp.zeros((MT, H), jnp.float32)

    @pl.when(vld_ref[i] == 1)
    def _go():
        x = xs_ref[...].astype(jnp.bfloat16)
        a = jax.lax.dot_general(x, w1_ref[0], (((1,), (1,)), ((), ())),
                                preferred_element_type=jnp.float32)
        b = jax.lax.dot_general(x, w3_ref[0], (((1,), (1,)), ((), ())),
                                preferred_element_type=jnp.float32)
        h1 = (a * jax.nn.sigmoid(a) * b).astype(jnp.bfloat16)
        o = jax.lax.dot_general(h1, w2_ref[0], (((1,), (1,)), ((), ())),
                                preferred_element_type=jnp.float32)
        sl = pl.ds(i * MT, MT)

        @pl.when(f == 0)
        def _init():
            acc_ref[sl, :] = o

        @pl.when(f > 0)
        def _acc():
            acc_ref[sl, :] += o

        @pl.when(f == NF - 1)
        def _emit():
            out_ref[...] = acc_ref[sl, :]


def _run_ffn(eot, vld, xs, W1b, W3b, W2b):
    grid_spec = pltpu.PrefetchScalarGridSpec(
        num_scalar_prefetch=2,
        grid=(NF, NTILES),
        in_specs=[
            pl.BlockSpec((MT, H), lambda f, i, eot, vld: (i, 0)),
            pl.BlockSpec((1, FFB, H), lambda f, i, eot, vld: (eot[i], f, 0)),
            pl.BlockSpec((1, FFB, H), lambda f, i, eot, vld: (eot[i], f, 0)),
            pl.BlockSpec((1, H, FFB), lambda f, i, eot, vld: (eot[i], 0, f)),
        ],
        out_specs=pl.BlockSpec((MT, H), lambda f, i, eot, vld: (i, 0)),
        scratch_shapes=[pltpu.VMEM((PADTOT, H), jnp.float32)],
    )
    return pl.pallas_call(
        _ffn_body,
        grid_spec=grid_spec,
        out_shape=jax.ShapeDtypeStruct((PADTOT, H), jnp.float32),
    )(eot, vld, xs, W1b, W3b, W2b)


def _gather2_body(os_hbm, p0_hbm, p1_hbm, y0_hbm, y1_hbm,
                  idx0_v, idx1_v, ra, rb, sema, semb):
    c = lax.axis_index("c")
    w = lax.axis_index("s")
    g = w * NC + c
    tpw = S // NW
    gb = g * tpw
    hh = tpw // 2
    pltpu.sync_copy(p0_hbm.at[pl.ds(gb, tpw)], idx0_v)
    pltpu.sync_copy(p1_hbm.at[pl.ds(gb, tpw)], idx1_v)
    d0 = pltpu.async_copy(os_hbm.at[idx0_v.at[pl.ds(0, hh)]], ra, sema)
    d1 = pltpu.async_copy(os_hbm.at[idx0_v.at[pl.ds(hh, hh)]], rb, semb)
    d0.wait()
    pltpu.sync_copy(ra, y0_hbm.at[pl.ds(gb, hh)])
    d2 = pltpu.async_copy(os_hbm.at[idx1_v.at[pl.ds(0, hh)]], ra, sema)
    d1.wait()
    pltpu.sync_copy(rb, y0_hbm.at[pl.ds(gb + hh, hh)])
    d3 = pltpu.async_copy(os_hbm.at[idx1_v.at[pl.ds(hh, hh)]], rb, semb)
    d2.wait()
    pltpu.sync_copy(ra, y1_hbm.at[pl.ds(gb, hh)])
    d3.wait()
    pltpu.sync_copy(rb, y1_hbm.at[pl.ds(gb + hh, hh)])


def _run_gather2(out_sorted, pos0, pos1):
    mesh = plsc.VectorSubcoreMesh(core_axis_name="c", subcore_axis_name="s",
                                  num_cores=NC, num_subcores=NS)
    f = pl.kernel(
        _gather2_body,
        out_type=[
            jax.ShapeDtypeStruct((S, H), jnp.float32),
            jax.ShapeDtypeStruct((S, H), jnp.float32),
        ],
        mesh=mesh,
        scratch_types=[
            pltpu.VMEM((S // NW,), jnp.int32),
            pltpu.VMEM((S // NW,), jnp.int32),
            pltpu.VMEM((S // NW // 2, H), jnp.float32),
            pltpu.VMEM((S // NW // 2, H), jnp.float32),
            pltpu.SemaphoreType.DMA,
            pltpu.SemaphoreType.DMA,
        ],
    )
    return f(out_sorted, pos0, pos1)


def _comb_body(ws_ref, y0_ref, y1_ref, out_ref):
    ws = ws_ref[...]
    out_ref[...] = (y0_ref[...] * ws[:, 0:1] + y1_ref[...] * ws[:, 1:2])


def _run_comb(wslots, y0, y1):
    nt = S // TM
    return pl.pallas_call(
        _comb_body,
        grid=(nt,),
        in_specs=[
            pl.BlockSpec((TM, 8), lambda t: (t, 0)),
            pl.BlockSpec((TM, H), lambda t: (t, 0)),
            pl.BlockSpec((TM, H), lambda t: (t, 0)),
        ],
        out_specs=pl.BlockSpec((TM, H), lambda t: (t, 0)),
        out_shape=jax.ShapeDtypeStruct((S, H), jnp.float32),
    )(wslots, y0, y1)


def kernel(hidden_states, gate_w, gate2_w, W1, W2, W3):
    x = hidden_states.reshape(-1, H)
    gpad = jnp.zeros((EPAD, H), jnp.float32).at[:NEXP].set(
        jnp.concatenate([gate_w, gate2_w], axis=0))

    logits16, eslots, wslots, pos8, eotvld = _run_router(x, gpad)

    sc_flat = jnp.concatenate([pos8[:, 2], pos8[:, 3]])
    tids = jnp.arange(A, dtype=jnp.int32) % S
    xs = _run_dispatch(sc_flat, tids, x)

    W1b = W1.astype(jnp.bfloat16)
    W2b = W2.astype(jnp.bfloat16)
    W3b = W3.astype(jnp.bfloat16)
    eot = eotvld[0, :NTILES]
    vld = eotvld[1, :NTILES]
    out_sorted = _run_ffn(eot, vld, xs, W1b, W3b, W2b)

    y0, y1 = _run_gather2(out_sorted, pos8[:, 0], pos8[:, 1])
    final = _run_comb(wslots, y0, y1)
    return final.reshape(B, S, H), logits16[:, :NEXP]
